# Initial kernel scaffold; baseline (speedup 1.0000x reference)
import jax
import jax.numpy as jnp
from jax.experimental import pallas as pl


def kernel(X_real, X_imag, L_real, L_imag, weight, bias):
    raise NotImplementedError("write your pallas kernel here")


# speedup vs baseline: 2.1993x; 2.1993x over previous
import jax
import jax.numpy as jnp
from jax.experimental import pallas as pl

_BK = 512


def _magconv_body(xr_ref, xi_ref, w_ref, lr_ref, li_ref, bias_ref,
                  real_ref, imag_ref):
    i = pl.program_id(0)
    k = pl.program_id(1)

    @pl.when((i == 0) & (k == 0))
    def _init():
        real_ref[...] = jnp.zeros_like(real_ref)
        imag_ref[...] = jnp.zeros_like(imag_ref)

    w = w_ref[0]
    yr = jnp.dot(xr_ref[...], w, preferred_element_type=jnp.float32)
    yi = jnp.dot(xi_ref[...], w, preferred_element_type=jnp.float32)

    lr = lr_ref[0].astype(jnp.bfloat16)
    li = li_ref[0].astype(jnp.bfloat16)
    yrb = yr.astype(jnp.bfloat16)
    yib = yi.astype(jnp.bfloat16)

    real_ref[...] += (
        jnp.dot(lr, yrb, preferred_element_type=jnp.float32)
        - jnp.dot(li, yib, preferred_element_type=jnp.float32))
    imag_ref[...] += (
        jnp.dot(li, yrb, preferred_element_type=jnp.float32)
        + jnp.dot(lr, yib, preferred_element_type=jnp.float32))

    @pl.when((i == pl.num_programs(0) - 1) & (k == pl.num_programs(1) - 1))
    def _finish():
        real_ref[...] += bias_ref[...]
        imag_ref[...] += bias_ref[...]


def kernel(X_real, X_imag, L_real, L_imag, weight, bias):
    n, c = X_real.shape
    hops = L_real.shape[0]
    kb = n // _BK

    real, imag = pl.pallas_call(
        _magconv_body,
        grid=(hops, kb),
        in_specs=[
            pl.BlockSpec((_BK, c), lambda i, k: (k, 0)),
            pl.BlockSpec((_BK, c), lambda i, k: (k, 0)),
            pl.BlockSpec((1, c, c), lambda i, k: (i, 0, 0)),
            pl.BlockSpec((1, n, _BK), lambda i, k: (i, 0, k)),
            pl.BlockSpec((1, n, _BK), lambda i, k: (i, 0, k)),
            pl.BlockSpec((1, c), lambda i, k: (0, 0)),
        ],
        out_specs=[
            pl.BlockSpec((n, c), lambda i, k: (0, 0)),
            pl.BlockSpec((n, c), lambda i, k: (0, 0)),
        ],
        out_shape=[
            jax.ShapeDtypeStruct((n, c), jnp.float32),
            jax.ShapeDtypeStruct((n, c), jnp.float32),
        ],
    )(X_real, X_imag, weight, L_real, L_imag, bias)
    return (real, imag)


# speedup vs baseline: 2.2845x; 1.0387x over previous
import jax
import jax.numpy as jnp
from jax.experimental import pallas as pl
from jax.experimental.pallas import tpu as pltpu

_BM = 256


def _magconv_body(xr_ref, xi_ref, w_ref, lr_ref, li_ref, bias_ref,
                  real_ref, imag_ref, yr_s, yi_s):
    m = pl.program_id(0)

    @pl.when(m == 0)
    def _compute_y():
        xr = xr_ref[...].astype(jnp.bfloat16)
        xi = xi_ref[...].astype(jnp.bfloat16)
        for i in range(w_ref.shape[0]):
            wb = w_ref[i].astype(jnp.bfloat16)
            yr_s[i] = jnp.dot(xr, wb,
                              preferred_element_type=jnp.float32
                              ).astype(jnp.bfloat16)
            yi_s[i] = jnp.dot(xi, wb,
                              preferred_element_type=jnp.float32
                              ).astype(jnp.bfloat16)

    f32 = jnp.float32
    real = bias_ref[...].astype(f32)
    imag = bias_ref[...].astype(f32)
    for i in range(lr_ref.shape[0]):
        lr = lr_ref[i].astype(jnp.bfloat16)
        li = li_ref[i].astype(jnp.bfloat16)
        yr = yr_s[i]
        yi = yi_s[i]
        real += (jnp.dot(lr, yr, preferred_element_type=f32)
                 - jnp.dot(li, yi, preferred_element_type=f32))
        imag += (jnp.dot(li, yr, preferred_element_type=f32)
                 + jnp.dot(lr, yi, preferred_element_type=f32))
    real_ref[...] = real
    imag_ref[...] = imag


def kernel(X_real, X_imag, L_real, L_imag, weight, bias):
    n, c = X_real.shape
    hops = L_real.shape[0]
    mb = n // _BM

    real, imag = pl.pallas_call(
        _magconv_body,
        grid=(mb,),
        in_specs=[
            pl.BlockSpec((n, c), lambda m: (0, 0)),
            pl.BlockSpec((n, c), lambda m: (0, 0)),
            pl.BlockSpec((hops, c, c), lambda m: (0, 0, 0)),
            pl.BlockSpec((hops, _BM, n), lambda m: (0, m, 0)),
            pl.BlockSpec((hops, _BM, n), lambda m: (0, m, 0)),
            pl.BlockSpec((1, c), lambda m: (0, 0)),
        ],
        out_specs=[
            pl.BlockSpec((_BM, c), lambda m: (m, 0)),
            pl.BlockSpec((_BM, c), lambda m: (m, 0)),
        ],
        out_shape=[
            jax.ShapeDtypeStruct((n, c), jnp.float32),
            jax.ShapeDtypeStruct((n, c), jnp.float32),
        ],
        scratch_shapes=[
            pltpu.VMEM((hops, n, c), jnp.bfloat16),
            pltpu.VMEM((hops, n, c), jnp.bfloat16),
        ],
    )(X_real, X_imag, weight, L_real, L_imag, bias)
    return (real, imag)


# speedup vs baseline: 2.4679x; 1.0803x over previous
import jax
import jax.numpy as jnp
from jax.experimental import pallas as pl
from jax.experimental.pallas import tpu as pltpu

_BM = 256


def _magconv_body(xr_ref, xi_ref, w_ref, lr_ref, li_ref, bias_ref,
                  real_ref, imag_ref, sr_s, si_s):
    m = pl.program_id(0)
    c = xr_ref.shape[1]
    f32 = jnp.float32
    bf16 = jnp.bfloat16

    @pl.when(m == 0)
    def _compute_rhs():
        xr = xr_ref[...].astype(bf16)
        xi = xi_ref[...].astype(bf16)
        for i in range(w_ref.shape[0]):
            wb = w_ref[i].astype(bf16)
            yr = jnp.dot(xr, wb, preferred_element_type=f32).astype(bf16)
            yi = jnp.dot(xi, wb, preferred_element_type=f32).astype(bf16)
            sr_s[i] = jnp.concatenate([yr, yi], axis=1)
            si_s[i] = jnp.concatenate([-yi, yr], axis=1)

    acc = jnp.zeros(real_ref.shape[:1] + (2 * c,), f32)
    for i in range(lr_ref.shape[0]):
        lr = lr_ref[i].astype(bf16)
        li = li_ref[i].astype(bf16)
        acc += (jnp.dot(lr, sr_s[i], preferred_element_type=f32)
                + jnp.dot(li, si_s[i], preferred_element_type=f32))
    bias = bias_ref[...].astype(f32)
    real_ref[...] = acc[:, :c] + bias
    imag_ref[...] = acc[:, c:] + bias


def kernel(X_real, X_imag, L_real, L_imag, weight, bias):
    n, c = X_real.shape
    hops = L_real.shape[0]
    mb = n // _BM

    real, imag = pl.pallas_call(
        _magconv_body,
        grid=(mb,),
        in_specs=[
            pl.BlockSpec((n, c), lambda m: (0, 0)),
            pl.BlockSpec((n, c), lambda m: (0, 0)),
            pl.BlockSpec((hops, c, c), lambda m: (0, 0, 0)),
            pl.BlockSpec((hops, _BM, n), lambda m: (0, m, 0)),
            pl.BlockSpec((hops, _BM, n), lambda m: (0, m, 0)),
            pl.BlockSpec((1, c), lambda m: (0, 0)),
        ],
        out_specs=[
            pl.BlockSpec((_BM, c), lambda m: (m, 0)),
            pl.BlockSpec((_BM, c), lambda m: (m, 0)),
        ],
        out_shape=[
            jax.ShapeDtypeStruct((n, c), jnp.float32),
            jax.ShapeDtypeStruct((n, c), jnp.float32),
        ],
        scratch_shapes=[
            pltpu.VMEM((hops, n, 2 * c), jnp.bfloat16),
            pltpu.VMEM((hops, n, 2 * c), jnp.bfloat16),
        ],
    )(X_real, X_imag, weight, L_real, L_imag, bias)
    return (real, imag)


# speedup vs baseline: 2.4752x; 1.0030x over previous
import jax
import jax.numpy as jnp
from jax.experimental import pallas as pl
from jax.experimental.pallas import tpu as pltpu

_BM = 128


def _magconv_body(xr_ref, xi_ref, w_ref, lr_ref, li_ref, bias_ref,
                  real_ref, imag_ref, sr_s, si_s):
    m = pl.program_id(0)
    c = xr_ref.shape[1]
    f32 = jnp.float32
    bf16 = jnp.bfloat16

    @pl.when(m == 0)
    def _compute_rhs():
        xr = xr_ref[...].astype(bf16)
        xi = xi_ref[...].astype(bf16)
        for i in range(w_ref.shape[0]):
            wb = w_ref[i].astype(bf16)
            yr = jnp.dot(xr, wb, preferred_element_type=f32).astype(bf16)
            yi = jnp.dot(xi, wb, preferred_element_type=f32).astype(bf16)
            sr_s[i] = jnp.concatenate([yr, yi], axis=1)
            si_s[i] = jnp.concatenate([-yi, yr], axis=1)

    acc = jnp.zeros(real_ref.shape[:1] + (2 * c,), f32)
    for i in range(lr_ref.shape[0]):
        lr = lr_ref[i].astype(bf16)
        li = li_ref[i].astype(bf16)
        acc += (jnp.dot(lr, sr_s[i], preferred_element_type=f32)
                + jnp.dot(li, si_s[i], preferred_element_type=f32))
    bias = bias_ref[...].astype(f32)
    real_ref[...] = acc[:, :c] + bias
    imag_ref[...] = acc[:, c:] + bias


def kernel(X_real, X_imag, L_real, L_imag, weight, bias):
    n, c = X_real.shape
    hops = L_real.shape[0]
    mb = n // _BM

    real, imag = pl.pallas_call(
        _magconv_body,
        grid=(mb,),
        in_specs=[
            pl.BlockSpec((n, c), lambda m: (0, 0)),
            pl.BlockSpec((n, c), lambda m: (0, 0)),
            pl.BlockSpec((hops, c, c), lambda m: (0, 0, 0)),
            pl.BlockSpec((hops, _BM, n), lambda m: (0, m, 0)),
            pl.BlockSpec((hops, _BM, n), lambda m: (0, m, 0)),
            pl.BlockSpec((1, c), lambda m: (0, 0)),
        ],
        out_specs=[
            pl.BlockSpec((_BM, c), lambda m: (m, 0)),
            pl.BlockSpec((_BM, c), lambda m: (m, 0)),
        ],
        out_shape=[
            jax.ShapeDtypeStruct((n, c), jnp.float32),
            jax.ShapeDtypeStruct((n, c), jnp.float32),
        ],
        scratch_shapes=[
            pltpu.VMEM((hops, n, 2 * c), jnp.bfloat16),
            pltpu.VMEM((hops, n, 2 * c), jnp.bfloat16),
        ],
    )(X_real, X_imag, weight, L_real, L_imag, bias)
    return (real, imag)
